# baseline (speedup 1.0000x reference)
import functools

import jax
import jax.numpy as jnp
from jax import lax
from jax.experimental import pallas as pl
from jax.experimental.pallas import tpu as pltpu
from jax.experimental.pallas import tpu_sc as plsc

_HIDDEN = 768
_LANES = 16
_VECS = _HIDDEN // _LANES
_C = 16
_NBUF = 8
_LEAD = 6


def _make_kernel(batch: int, seq: int):
    info = plsc.get_sparse_core_info()
    nc, ns = info.num_cores, info.num_subcores
    nw = nc * ns
    pos_per_w = seq // nw
    n_chunks = batch // _C

    mesh = plsc.VectorSubcoreMesh(core_axis_name="c", subcore_axis_name="s")

    @functools.partial(
        pl.kernel,
        mesh=mesh,
        out_type=jax.ShapeDtypeStruct((batch, seq, _HIDDEN), jnp.float32),
        scratch_types=[
            pltpu.VMEM((batch,), jnp.int32),
            pltpu.VMEM((_HIDDEN,), jnp.float32),
            pltpu.VMEM((_NBUF, _C, _HIDDEN), jnp.float32),
        ]
        + [pltpu.SemaphoreType.DMA] * (2 * _NBUF),
    )
    def k(ids_t_hbm, token_hbm, pos_hbm, out_hbm, idx_v, pos_v, bufs, *sems):
        sem_g = sems[:_NBUF]
        sem_w = sems[_NBUF:]
        wid = lax.axis_index("s") * nc + lax.axis_index("c")

        def gather(j, b):
            pltpu.async_copy(
                token_hbm.at[idx_v.at[pl.ds(j * _C, _C)]], bufs.at[b], sem_g[b]
            )

        def gather_wait(j, b):
            pltpu.make_async_copy(
                token_hbm.at[idx_v.at[pl.ds(j * _C, _C)]], bufs.at[b], sem_g[b]
            ).wait()

        def write(j, b, s):
            pltpu.async_copy(
                bufs.at[b], out_hbm.at[pl.ds(j * _C, _C), s], sem_w[b]
            )

        def write_wait(j, b, s):
            pltpu.make_async_copy(
                bufs.at[b], out_hbm.at[pl.ds(j * _C, _C), s], sem_w[b]
            ).wait()

        for half in range(pos_per_w):
            s = wid * pos_per_w + half
            pltpu.sync_copy(ids_t_hbm.at[s], idx_v)
            pltpu.sync_copy(pos_hbm.at[s], pos_v)
            pv = tuple(pos_v[pl.ds(v * _LANES, _LANES)] for v in range(_VECS))

            for b in range(_LEAD):
                gather(b, b)

            def group(g, pv):
                for b in range(_NBUF):
                    j = _NBUF * g + b
                    gather_wait(j, b)

                    def row(r, pv):
                        for v in range(_VECS):
                            sl = pl.ds(v * _LANES, _LANES)
                            bufs[b, r, sl] = bufs[b, r, sl] + pv[v]
                        return pv

                    pv = lax.fori_loop(0, _C, row, pv)
                    write(j, b, s)

                    jn = j + _LEAD
                    bn = (b + _LEAD) % _NBUF

                    @pl.when(jn < n_chunks)
                    def _():
                        @pl.when(j >= _NBUF - _LEAD)
                        def _():
                            write_wait(jn - _NBUF, bn, s)

                        gather(jn, bn)

                return pv

            lax.fori_loop(0, n_chunks // _NBUF, group, pv)

            for b in range(_NBUF):
                write_wait(n_chunks - _NBUF + b, b, s)

    return k


def kernel(input_ids, token_table, pos_table):
    b, s = input_ids.shape
    ids_t = jnp.transpose(input_ids).astype(jnp.int32)
    return _make_kernel(b, s)(ids_t, token_table, pos_table)
